# baseline (speedup 1.0000x reference)
import functools

import jax
import jax.numpy as jnp
from jax import lax
from jax.experimental import pallas as pl
from jax.experimental.pallas import tpu as pltpu
from jax.experimental.pallas import tpu_sc as plsc

MAX_ATOMIC_NUM = 100
EMB_SIZE = 128
N_ATOMS = 100000

NC = 2
NS = 16
NW = NC * NS

CHUNK = 128
GCHUNKS = 3
GROUP = GCHUNKS * CHUNK
NGROUPS = 8
B_PER_W = NGROUPS * GROUP + CHUNK
EXTRA_OFF = NGROUPS * GROUP
LAST_W = NW - 1
LAST_START = LAST_W * B_PER_W
LAST_ROWS = 800
LAST_NGROUPS = 2
TAIL = 32
TAIL_OFF = N_ATOMS - TAIL


def _emb_body(table_hbm, z_hbm, out_hbm, table_s, idx_v, buf_a, buf_b,
              idx_t, rows_t, isem, ga, gb, wa, wb, tsem):
    s = lax.axis_index("s")
    c = lax.axis_index("c")
    wid = s * NC + c
    start = wid * B_PER_W

    @pl.when(wid < LAST_W)
    def _stage_idx_full():
        pltpu.async_copy(z_hbm.at[pl.ds(start, B_PER_W)], idx_v, isem).wait()

    @pl.when(wid == LAST_W)
    def _stage_idx_short():
        pltpu.async_copy(z_hbm.at[pl.ds(LAST_START, LAST_ROWS)],
                         idx_v.at[pl.ds(0, LAST_ROWS)], isem).wait()

    @pl.when(s == 0)
    def _stage_table():
        pltpu.sync_copy(table_hbm, table_s)

    plsc.subcore_barrier()

    bufs = (buf_a, buf_b)
    gsems = (ga, gb)
    wsems = (wa, wb)
    ng = jnp.where(wid == LAST_W, LAST_NGROUPS, NGROUPS)

    def fire_gathers(g, half):
        loc = pl.multiple_of(g * GROUP, CHUNK)
        for k in range(GCHUNKS):
            pltpu.async_copy(
                table_s.at[idx_v.at[pl.ds(loc + k * CHUNK, CHUNK)]],
                bufs[half].at[pl.ds(k * CHUNK, CHUNK)], gsems[half])

    def gathers_done(half):
        pltpu.make_async_copy(out_hbm.at[pl.ds(0, GROUP)], bufs[half],
                              gsems[half]).wait()

    def write_group(g, half):
        off = pl.multiple_of(start + g * GROUP, CHUNK)
        return pltpu.make_async_copy(
            bufs[half], out_hbm.at[pl.ds(off, GROUP)], wsems[half])

    @pl.loop(0, ng, step=2)
    def _pair(g0):
        for half in range(2):
            g = g0 + half

            @pl.when(g0 > 0)
            def _buffer_free(g=g, half=half):
                write_group(g - 2, half).wait()

            fire_gathers(g, half)
            if half == 0:
                @pl.when(g0 > 0)
                def _retire_prev(g=g):
                    gathers_done(1)
                    write_group(g - 1, 1).start()
            else:
                gathers_done(0)
                write_group(g - 1, 0).start()

    gathers_done(1)
    write_group(ng - 1, 1).start()

    @pl.when(wid < LAST_W)
    def _extra_chunk():
        write_group(NGROUPS - 2, 0).wait()
        pltpu.async_copy(
            table_s.at[idx_v.at[pl.ds(EXTRA_OFF, CHUNK)]],
            buf_a.at[pl.ds(0, CHUNK)], ga).wait()
        pltpu.async_copy(
            buf_a.at[pl.ds(0, CHUNK)],
            out_hbm.at[pl.ds(start + EXTRA_OFF, CHUNK)], wa).wait()

    @pl.when(wid == LAST_W)
    def _short_drain_a():
        write_group(0, 0).wait()

    @pl.when(wid == LAST_W)
    def _tail():
        pltpu.sync_copy(z_hbm.at[pl.ds(TAIL_OFF, TAIL)], idx_t)
        pltpu.async_copy(table_s.at[idx_t], rows_t, tsem).wait()
        pltpu.sync_copy(rows_t, out_hbm.at[pl.ds(TAIL_OFF, TAIL)])

    write_group(ng - 1, 1).wait()


_emb = functools.partial(
    pl.kernel,
    mesh=plsc.VectorSubcoreMesh(core_axis_name="c", subcore_axis_name="s"),
    out_type=jax.ShapeDtypeStruct((N_ATOMS, EMB_SIZE), jnp.float32),
    scratch_types=[
        pltpu.VMEM_SHARED((MAX_ATOMIC_NUM, EMB_SIZE), jnp.float32),
        pltpu.VMEM((B_PER_W,), jnp.int32),
        pltpu.VMEM((GROUP, EMB_SIZE), jnp.float32),
        pltpu.VMEM((GROUP, EMB_SIZE), jnp.float32),
        pltpu.VMEM((TAIL,), jnp.int32),
        pltpu.VMEM((TAIL, EMB_SIZE), jnp.float32),
    ] + [pltpu.SemaphoreType.DMA] * 6,
)(_emb_body)


def kernel(Z, table):
    return _emb(table, jnp.asarray(Z, jnp.int32))
